# baseline (speedup 1.0000x reference)
import functools
import math

import jax
import jax.numpy as jnp
from jax import lax
from jax.experimental import pallas as pl
from jax.experimental.pallas import tpu as pltpu
from jax.experimental.pallas import tpu_sc as plsc

N = 10000
E = 320000
S = 128
V = 16
RBF_DIM = 16
RBF_DMAX = 15.0
NORM = 10.0
H = V + 1

TBL = 256
MSG = 176

EB = 2560
NB = 2000


def _sigmoid(x):
    return 1.0 / (1.0 + jnp.exp(-x))


def _pre_kernel(sf_ref, cf_ref, pos_ref, wouts_ref, wh_ref, out_ref):
    pre_s = jnp.dot(sf_ref[...], wouts_ref[...], preferred_element_type=jnp.float32)
    whv = wh_ref[:V, :]
    nb = sf_ref.shape[0]
    blocks = []
    for c in range(3):
        blocks.append(jnp.dot(cf_ref[:, c * V:(c + 1) * V], whv,
                              preferred_element_type=jnp.float32))
        blocks.append(jnp.zeros((nb, 32 - H), jnp.float32))
    blocks.append(jnp.zeros((nb, 32), jnp.float32))
    pre_v = jnp.concatenate(blocks, axis=1)
    au = jax.lax.bitcast_convert_type(pre_s.astype(jnp.bfloat16),
                                      jnp.uint16).astype(jnp.int32)
    bu = jax.lax.bitcast_convert_type(pre_v.astype(jnp.bfloat16),
                                      jnp.uint16).astype(jnp.int32)
    out_ref[...] = au | (bu << 16)


def _build_table(scalar_feats, cf_packed, positions, w_out_s, wh):
    return pl.pallas_call(
        _pre_kernel,
        grid=(N // NB,),
        in_specs=[
            pl.BlockSpec((NB, S), lambda i: (i, 0)),
            pl.BlockSpec((NB, 3 * V), lambda i: (i, 0)),
            pl.BlockSpec((NB, 3), lambda i: (i, 0)),
            pl.BlockSpec((S, S), lambda i: (0, 0)),
            pl.BlockSpec((H, H), lambda i: (0, 0)),
        ],
        out_specs=pl.BlockSpec((NB, S), lambda i: (i, 0)),
        out_shape=jax.ShapeDtypeStruct((N, S), jnp.int32),
    )(scalar_feats, cf_packed, positions, w_out_s, wh)


def _edge_kernel(g_ref, pd_ref, ones3_ref, ub_ref, wh16t_ref, sumb_ref,
                 wrbf_ref, wshp_ref, bout_ref, wgate_ref, bgate_ref, g3_ref,
                 wub_ref, out_ref, outv_ref):
    gi = g_ref[...]
    pre_s = jax.lax.bitcast_convert_type(gi << 16, jnp.float32)
    pre_v = jax.lax.bitcast_convert_type(gi & jnp.int32(-65536), jnp.float32)
    xd = jnp.transpose(pd_ref[...], (1, 0))[:, :3]
    d2b = jnp.dot(xd * xd, ones3_ref[...],
                  preferred_element_type=jnp.float32) + 1e-8
    inv_b = jax.lax.rsqrt(d2b)
    xdb = jnp.dot(xd, ub_ref[...], preferred_element_type=jnp.float32)
    unitb = xdb * inv_b

    j = jax.lax.broadcasted_iota(jnp.int32, (1, RBF_DIM), 1).astype(jnp.float32)
    sigma = RBF_DMAX / RBF_DIM
    dist16 = d2b[:, :RBF_DIM] * inv_b[:, :RBF_DIM]
    z = (dist16 - j * (RBF_DMAX / (RBF_DIM - 1))) / sigma
    rbf = jnp.exp(-(z * z))

    vh = pre_v + unitb * wh16t_ref[...]
    ssq = jnp.dot(vh * vh, sumb_ref[...], preferred_element_type=jnp.float32)
    sh = jnp.sqrt(ssq + 1e-8)

    lin = (pre_s
           + jnp.dot(rbf, wrbf_ref[...], preferred_element_type=jnp.float32)
           + jnp.dot(sh, wshp_ref[...], preferred_element_type=jnp.float32)
           + bout_ref[...])
    msg_s = lin * _sigmoid(lin)
    gate = _sigmoid(jnp.dot(msg_s, wgate_ref[...],
                            preferred_element_type=jnp.float32) + bgate_ref[...])
    gatet = jnp.dot(gate, g3_ref[...], preferred_element_type=jnp.float32)
    vu = jnp.dot(vh, wub_ref[...], preferred_element_type=jnp.float32)
    out_ref[...] = msg_s
    outv_ref[...] = gatet * vu


def _edge_stage(g, pd, p):
    pm = p['msg']
    wu = pm['Wu']
    ones3 = jnp.ones((3, S), jnp.float32)
    ub = jnp.zeros((3, S), jnp.float32)
    wh16t = jnp.zeros((1, S), jnp.float32)
    sumb = jnp.zeros((S, 32), jnp.float32)
    wub = jnp.zeros((S, S), jnp.float32)
    for c in range(3):
        ub = ub.at[c, 32 * c:32 * c + H].set(1.0)
        wh16t = wh16t.at[0, 32 * c:32 * c + H].set(pm['Wh'][V])
        sumb = sumb.at[32 * c:32 * c + 32, :].set(jnp.eye(32, dtype=jnp.float32))
        wub = wub.at[32 * c:32 * c + H, 16 * c:16 * c + V].set(wu)
    wshp = jnp.zeros((32, S), jnp.float32).at[:H, :].set(pm['W_out'][S + RBF_DIM:])
    g3 = jnp.zeros((V, S), jnp.float32)
    for c in range(3):
        g3 = g3.at[:, 16 * c:16 * c + V].set(jnp.eye(V, dtype=jnp.float32))
    return pl.pallas_call(
        _edge_kernel,
        grid=(E // EB,),
        in_specs=[
            pl.BlockSpec((EB, S), lambda i: (i, 0)),
            pl.BlockSpec((8, EB), lambda i: (0, i)),
            pl.BlockSpec((3, S), lambda i: (0, 0)),
            pl.BlockSpec((3, S), lambda i: (0, 0)),
            pl.BlockSpec((1, S), lambda i: (0, 0)),
            pl.BlockSpec((S, 32), lambda i: (0, 0)),
            pl.BlockSpec((RBF_DIM, S), lambda i: (0, 0)),
            pl.BlockSpec((32, S), lambda i: (0, 0)),
            pl.BlockSpec((1, S), lambda i: (0, 0)),
            pl.BlockSpec((S, V), lambda i: (0, 0)),
            pl.BlockSpec((1, V), lambda i: (0, 0)),
            pl.BlockSpec((V, S), lambda i: (0, 0)),
            pl.BlockSpec((S, S), lambda i: (0, 0)),
        ],
        out_specs=[
            pl.BlockSpec((EB, S), lambda i: (i, 0)),
            pl.BlockSpec((EB, S), lambda i: (i, 0)),
        ],
        out_shape=[
            jax.ShapeDtypeStruct((E, S), jnp.float32),
            jax.ShapeDtypeStruct((E, S), jnp.float32),
        ],
    )(g, pd, ones3, ub, wh16t, sumb, pm['W_out'][S:S + RBF_DIM], wshp,
      pm['b_out'][None, :], pm['W_gate'], pm['b_gate'][None, :], g3, wub)


def _node_kernel(sf_ref, cf_ref, agg_ref, wh_ref, wu_ref, wouts_ref, woutv_ref,
                 bout_ref, wgate_ref, bgate_ref, s_out_ref, v_out_ref):
    agg_s = agg_ref[0] * (1.0 / NORM)
    agg_v = [agg_ref[1, :, c * V:(c + 1) * V] * (1.0 / NORM) for c in range(3)]

    mu = jnp.mean(agg_s, axis=1, keepdims=True)
    var = jnp.mean((agg_s - mu) ** 2, axis=1, keepdims=True)
    nf = (agg_s - mu) / jnp.sqrt(var + 1e-5)
    vsq = jnp.maximum(agg_v[0] ** 2 + agg_v[1] ** 2 + agg_v[2] ** 2, 1e-8)
    vn = jnp.sqrt(jnp.mean(vsq, axis=1, keepdims=True))
    inv_vn = 1.0 / vn
    s1 = sf_ref[...] + nf
    v1 = [cf_ref[:, c * V:(c + 1) * V] + agg_v[c] * inv_vn for c in range(3)]

    vh = [jnp.dot(v1[c], wh_ref[...], preferred_element_type=jnp.float32)
          for c in range(3)]
    ssq = jnp.maximum(vh[0] ** 2 + vh[1] ** 2 + vh[2] ** 2, 1e-8)
    sh = jnp.sqrt(ssq)
    lin = (jnp.dot(s1, wouts_ref[...], preferred_element_type=jnp.float32)
           + jnp.dot(sh, woutv_ref[...], preferred_element_type=jnp.float32)
           + bout_ref[...])
    f_out = lin * _sigmoid(lin)
    gate = _sigmoid(jnp.dot(f_out, wgate_ref[...],
                            preferred_element_type=jnp.float32) + bgate_ref[...])
    uv = [gate * jnp.dot(vh[c], wu_ref[...], preferred_element_type=jnp.float32)
          for c in range(3)]

    s2p = s1 + f_out
    v2p = [v1[c] + uv[c] for c in range(3)]
    mu2 = jnp.mean(s2p, axis=1, keepdims=True)
    var2 = jnp.mean((s2p - mu2) ** 2, axis=1, keepdims=True)
    s_out_ref[...] = (s2p - mu2) / jnp.sqrt(var2 + 1e-5)
    vsq2 = jnp.maximum(v2p[0] ** 2 + v2p[1] ** 2 + v2p[2] ** 2, 1e-8)
    inv_vn2 = 1.0 / jnp.sqrt(jnp.mean(vsq2, axis=1, keepdims=True))
    v_out_ref[...] = jnp.concatenate([v2p[c] * inv_vn2 for c in range(3)], axis=1)


def _node_stage(scalar_feats, cf_packed, agg, p):
    pu = p['upd']
    return pl.pallas_call(
        _node_kernel,
        grid=(N // NB,),
        in_specs=[
            pl.BlockSpec((NB, S), lambda i: (i, 0)),
            pl.BlockSpec((NB, 3 * V), lambda i: (i, 0)),
            pl.BlockSpec((2, NB, S), lambda i: (0, i, 0)),
            pl.BlockSpec((V, V), lambda i: (0, 0)),
            pl.BlockSpec((V, V), lambda i: (0, 0)),
            pl.BlockSpec((S, S), lambda i: (0, 0)),
            pl.BlockSpec((V, S), lambda i: (0, 0)),
            pl.BlockSpec((1, S), lambda i: (0, 0)),
            pl.BlockSpec((S, V), lambda i: (0, 0)),
            pl.BlockSpec((1, V), lambda i: (0, 0)),
        ],
        out_specs=[
            pl.BlockSpec((NB, S), lambda i: (i, 0)),
            pl.BlockSpec((NB, 3 * V), lambda i: (i, 0)),
        ],
        out_shape=[
            jax.ShapeDtypeStruct((N, S), jnp.float32),
            jax.ShapeDtypeStruct((N, 3 * V), jnp.float32),
        ],
    )(scalar_feats, cf_packed, agg, pu['Wh'], pu['Wu'], pu['W_out'][:S],
      pu['W_out'][S:], pu['b_out'][None, :], pu['W_gate'], pu['b_gate'][None, :])


_GCH = 128
_GNCHUNK = E // _GCH


def _sc_gather_body(tbl_hbm, pos_hbm, src_hbm, dst_hbm, out_hbm, pd_hbm,
                    idx_v, idx2_v, rows_v, pd_v, pos_v, lsem, gsem, wsem):
    cid = lax.axis_index("c")
    sid = lax.axis_index("s")
    wid = sid * _SC_NC + cid
    nfull = _GNCHUNK // _NW
    nch = nfull + jnp.where(wid < _GNCHUNK - nfull * _NW, 1, 0)

    pltpu.sync_copy(pos_hbm, pos_v)
    for b in range(2):
        for r in range(3, 8):
            for k in range(_GCH // 16):
                pd_v[b, r, pl.ds(k * 16, 16)] = jnp.zeros((16,), jnp.float32)

    def _issue_loads(i, b):
        off = (i * _NW + wid) * _GCH
        pltpu.async_copy(src_hbm.at[pl.ds(off, _GCH)], idx_v.at[b], lsem.at[b])
        pltpu.async_copy(dst_hbm.at[pl.ds(off, _GCH)], idx2_v.at[b], lsem.at[b])

    _issue_loads(0, 0)

    def _step(i, _):
        for b in range(2):
            ii = i * 2 + b
            pltpu.make_async_copy(src_hbm.at[pl.ds(0, _GCH)], idx_v.at[b], lsem.at[b]).wait()
            pltpu.make_async_copy(dst_hbm.at[pl.ds(0, _GCH)], idx2_v.at[b], lsem.at[b]).wait()

            @pl.when(ii >= 2)
            def _():
                pltpu.make_async_copy(tbl_hbm.at[pl.ds(0, _GCH)], rows_v.at[b], wsem.at[b]).wait()
                pltpu.make_async_copy(pd_hbm.at[:, pl.ds(0, _GCH)], pd_v.at[b], wsem.at[b]).wait()

            pltpu.async_copy(tbl_hbm.at[idx_v.at[b]], rows_v.at[b], gsem.at[b])

            @pl.when(ii + 1 < nch)
            def _():
                _issue_loads(ii + 1, (b + 1) % 2)

            for k in range(_GCH // 16):
                d4 = idx2_v[b, pl.ds(k * 16, 16)] * 4
                s4 = idx_v[b, pl.ds(k * 16, 16)] * 4
                for c in range(3):
                    pd_v[b, c, pl.ds(k * 16, 16)] = (
                        plsc.load_gather(pos_v, [d4 + c])
                        - plsc.load_gather(pos_v, [s4 + c]))
            pltpu.make_async_copy(tbl_hbm.at[pl.ds(0, _GCH)], rows_v.at[b], gsem.at[b]).wait()
            off = (ii * _NW + wid) * _GCH
            pltpu.async_copy(rows_v.at[b], out_hbm.at[pl.ds(off, _GCH)], wsem.at[b])
            pltpu.async_copy(pd_v.at[b], pd_hbm.at[:, pl.ds(off, _GCH)], wsem.at[b])
        return _

    lax.fori_loop(0, nch // 2, _step, None)

    @pl.when(nch % 2 == 1)
    def _():
        ii = nch - 1
        b = 0
        pltpu.make_async_copy(src_hbm.at[pl.ds(0, _GCH)], idx_v.at[b], lsem.at[b]).wait()
        pltpu.make_async_copy(dst_hbm.at[pl.ds(0, _GCH)], idx2_v.at[b], lsem.at[b]).wait()

        @pl.when(ii >= 2)
        def _():
            pltpu.make_async_copy(tbl_hbm.at[pl.ds(0, _GCH)], rows_v.at[b], wsem.at[b]).wait()
            pltpu.make_async_copy(pd_hbm.at[:, pl.ds(0, _GCH)], pd_v.at[b], wsem.at[b]).wait()

        pltpu.async_copy(tbl_hbm.at[idx_v.at[b]], rows_v.at[b], gsem.at[b])
        for k in range(_GCH // 16):
            d4 = idx2_v[b, pl.ds(k * 16, 16)] * 4
            s4 = idx_v[b, pl.ds(k * 16, 16)] * 4
            for c in range(3):
                pd_v[b, c, pl.ds(k * 16, 16)] = (
                    plsc.load_gather(pos_v, [d4 + c])
                    - plsc.load_gather(pos_v, [s4 + c]))
        pltpu.make_async_copy(tbl_hbm.at[pl.ds(0, _GCH)], rows_v.at[b], gsem.at[b]).wait()
        off = (ii * _NW + wid) * _GCH
        pltpu.async_copy(rows_v.at[b], out_hbm.at[pl.ds(off, _GCH)], wsem.at[b])
        pltpu.async_copy(pd_v.at[b], pd_hbm.at[:, pl.ds(off, _GCH)], wsem.at[b])

    @pl.when(nch >= 2)
    def _():
        pltpu.make_async_copy(tbl_hbm.at[pl.ds(0, _GCH)], rows_v.at[1], wsem.at[1]).wait()
        pltpu.make_async_copy(pd_hbm.at[:, pl.ds(0, _GCH)], pd_v.at[1], wsem.at[1]).wait()

    @pl.when(nch >= 1)
    def _():
        pltpu.make_async_copy(tbl_hbm.at[pl.ds(0, _GCH)], rows_v.at[0], wsem.at[0]).wait()
        pltpu.make_async_copy(pd_hbm.at[:, pl.ds(0, _GCH)], pd_v.at[0], wsem.at[0]).wait()


def _sc_gather(table, posflat, src, dst):
    mesh = plsc.VectorSubcoreMesh(core_axis_name="c", subcore_axis_name="s")
    f = functools.partial(
        pl.kernel,
        mesh=mesh,
        out_type=[
            jax.ShapeDtypeStruct((E, S), jnp.int32),
            jax.ShapeDtypeStruct((8, E), jnp.float32),
        ],
        scratch_types=[
            pltpu.VMEM((2, _GCH), jnp.int32),
            pltpu.VMEM((2, _GCH), jnp.int32),
            pltpu.VMEM((2, _GCH, S), jnp.int32),
            pltpu.VMEM((2, 8, _GCH), jnp.float32),
            pltpu.VMEM((4 * N,), jnp.float32),
            pltpu.SemaphoreType.DMA((2,)),
            pltpu.SemaphoreType.DMA((2,)),
            pltpu.SemaphoreType.DMA((2,)),
        ],
        compiler_params=pltpu.CompilerParams(needs_layout_passes=False),
    )(_sc_gather_body)
    return f(table, posflat, src, dst)


_SC_NC = 2
_SC_NS = 16
_NW = _SC_NC * _SC_NS
_CH = 80
_EPW = E // _NW
_EPC = E // _SC_NS


def _sc_scatter_body(msga_hbm, msgb_hbm, dst_hbm, out_hbm, idx_v, rows_v,
                     zero_v, acc_sh, lsem, asem):
    cid = lax.axis_index("c")
    sid = lax.axis_index("s")

    for r in range(16):
        for k in range(S // 16):
            zero_v[r, pl.ds(k * 16, 16)] = jnp.zeros((16,), jnp.float32)

    def _zero(j, _):
        pltpu.sync_copy(zero_v, acc_sh.at[pl.ds((j * _SC_NS + sid) * 16, 16)])
        return _

    lax.fori_loop(0, N // (16 * _SC_NS), _zero, None)
    nblk = N // 16
    done = (N // (16 * _SC_NS)) * _SC_NS

    @pl.when(sid < nblk - done)
    def _():
        pltpu.sync_copy(zero_v, acc_sh.at[pl.ds((done + sid) * 16, 16)])

    plsc.subcore_barrier()

    base = sid * _EPC
    nchunk = _EPC // _CH

    def _issue_loads(msg_hbm, c, b):
        off = base + c * _CH
        pltpu.async_copy(dst_hbm.at[pl.ds(off, _CH)], idx_v.at[b], lsem.at[b])
        pltpu.async_copy(msg_hbm.at[pl.ds(off, _CH)], rows_v.at[b], lsem.at[b])

    def _drain_loads(msg_hbm, b):
        pltpu.make_async_copy(dst_hbm.at[pl.ds(0, _CH)], idx_v.at[b], lsem.at[b]).wait()
        pltpu.make_async_copy(msg_hbm.at[pl.ds(0, _CH)], rows_v.at[b], lsem.at[b]).wait()

    def _drain_add(msg_hbm, b):
        pltpu.make_async_copy(msg_hbm.at[pl.ds(0, _CH)], rows_v.at[b], asem.at[b]).wait()

    def _run(msg_hbm):
        _issue_loads(msg_hbm, 0, 0)

        def _step(i, _):
            for b in range(2):
                c = i * 2 + b
                nb = (b + 1) % 2
                _drain_loads(msg_hbm, b)
                pltpu.async_copy(rows_v.at[b], acc_sh.at[idx_v.at[b]],
                                 asem.at[b], add=True)

                @pl.when(c >= 1)
                def _():
                    _drain_add(msg_hbm, nb)

                @pl.when(c + 1 < nchunk)
                def _():
                    _issue_loads(msg_hbm, c + 1, nb)

            return _

        lax.fori_loop(0, nchunk // 2, _step, None)
        _drain_add(msg_hbm, 1)

    @pl.when(cid == 0)
    def _():
        _run(msga_hbm)

    @pl.when(cid == 1)
    def _():
        _run(msgb_hbm)

    plsc.subcore_barrier()

    rows_per = 632

    @pl.when(sid < _SC_NS - 1)
    def _():
        r0 = sid * rows_per
        pltpu.sync_copy(acc_sh.at[pl.ds(r0, rows_per)],
                        out_hbm.at[cid].at[pl.ds(r0, rows_per)])

    @pl.when(sid == _SC_NS - 1)
    def _():
        r0 = (_SC_NS - 1) * rows_per
        pltpu.sync_copy(acc_sh.at[pl.ds(r0, N - r0)],
                        out_hbm.at[cid].at[pl.ds(r0, N - r0)])


def _sc_scatter(msgs_a, msgs_b, dst):
    mesh = plsc.VectorSubcoreMesh(core_axis_name="c", subcore_axis_name="s")
    f = functools.partial(
        pl.kernel,
        mesh=mesh,
        out_type=jax.ShapeDtypeStruct((_SC_NC, N, S), jnp.float32),
        scratch_types=[
            pltpu.VMEM((2, _CH), jnp.int32),
            pltpu.VMEM((2, _CH, S), jnp.float32),
            pltpu.VMEM((16, S), jnp.float32),
            pltpu.VMEM_SHARED((N, S), jnp.float32),
            pltpu.SemaphoreType.DMA((2,)),
            pltpu.SemaphoreType.DMA((2,)),
        ],
    )(_sc_scatter_body)
    return f(msgs_a, msgs_b, dst)


def kernel(scalar_feats, coord_feats, positions, edge_index, params):
    src = edge_index[0]
    dst = edge_index[1]
    cf_packed = jnp.transpose(coord_feats, (0, 2, 1)).reshape(N, 3 * V)

    table = _build_table(scalar_feats, cf_packed, positions,
                         params['msg']['W_out'][:S], params['msg']['Wh'])

    posflat = jnp.pad(positions, ((0, 0), (0, 1))).reshape(-1)
    g, pd = _sc_gather(table, posflat, src, dst)
    msgs_a, msgs_b = _edge_stage(g, pd, params)
    agg = _sc_scatter(msgs_a, msgs_b, dst)

    s2, v2p = _node_stage(scalar_feats, cf_packed, agg, params)
    v2 = jnp.transpose(v2p.reshape(N, 3, V), (0, 2, 1))
    return s2, v2


# speedup vs baseline: 1.0400x; 1.0400x over previous
import functools
import math

import jax
import jax.numpy as jnp
from jax import lax
from jax.experimental import pallas as pl
from jax.experimental.pallas import tpu as pltpu
from jax.experimental.pallas import tpu_sc as plsc

N = 10000
E = 320000
S = 128
V = 16
RBF_DIM = 16
RBF_DMAX = 15.0
NORM = 10.0
H = V + 1

TBL = 256
MSG = 176

EB = 2560
NB = 2000


def _sigmoid(x):
    return 1.0 / (1.0 + jnp.exp(-x))


def _pre_kernel(sf_ref, cf_ref, pos_ref, wouts_ref, wh_ref, out_ref):
    pre_s = jnp.dot(sf_ref[...], wouts_ref[...], preferred_element_type=jnp.float32)
    whv = wh_ref[:V, :]
    nb = sf_ref.shape[0]
    blocks = []
    for c in range(3):
        blocks.append(jnp.dot(cf_ref[:, c * V:(c + 1) * V], whv,
                              preferred_element_type=jnp.float32))
        blocks.append(jnp.zeros((nb, 32 - H), jnp.float32))
    blocks.append(jnp.zeros((nb, 32), jnp.float32))
    pre_v = jnp.concatenate(blocks, axis=1)
    au = jax.lax.bitcast_convert_type(pre_s.astype(jnp.bfloat16),
                                      jnp.uint16).astype(jnp.int32)
    bu = jax.lax.bitcast_convert_type(pre_v.astype(jnp.bfloat16),
                                      jnp.uint16).astype(jnp.int32)
    out_ref[...] = au | (bu << 16)


def _build_table(scalar_feats, cf_packed, positions, w_out_s, wh):
    return pl.pallas_call(
        _pre_kernel,
        grid=(N // NB,),
        in_specs=[
            pl.BlockSpec((NB, S), lambda i: (i, 0)),
            pl.BlockSpec((NB, 3 * V), lambda i: (i, 0)),
            pl.BlockSpec((NB, 3), lambda i: (i, 0)),
            pl.BlockSpec((S, S), lambda i: (0, 0)),
            pl.BlockSpec((H, H), lambda i: (0, 0)),
        ],
        out_specs=pl.BlockSpec((NB, S), lambda i: (i, 0)),
        out_shape=jax.ShapeDtypeStruct((N, S), jnp.int32),
    )(scalar_feats, cf_packed, positions, w_out_s, wh)


def _edge_kernel(g_ref, pd_ref, ones3_ref, ub_ref, wh16t_ref, sumb_ref,
                 wrbf_ref, wshp_ref, bout_ref, wgate_ref, bgate_ref, g3_ref,
                 wub_ref, out_ref, outv_ref):
    gi = g_ref[...]
    pre_s = jax.lax.bitcast_convert_type(gi << 16, jnp.float32)
    pre_v = jax.lax.bitcast_convert_type(gi & jnp.int32(-65536), jnp.float32)
    xd = jnp.transpose(pd_ref[...], (1, 0))[:, :3]
    d2b = jnp.dot(xd * xd, ones3_ref[...],
                  preferred_element_type=jnp.float32) + 1e-8
    inv_b = jax.lax.rsqrt(d2b)
    xdb = jnp.dot(xd, ub_ref[...], preferred_element_type=jnp.float32)
    unitb = xdb * inv_b

    j = jax.lax.broadcasted_iota(jnp.int32, (1, RBF_DIM), 1).astype(jnp.float32)
    sigma = RBF_DMAX / RBF_DIM
    dist16 = d2b[:, :RBF_DIM] * inv_b[:, :RBF_DIM]
    z = (dist16 - j * (RBF_DMAX / (RBF_DIM - 1))) / sigma
    rbf = jnp.exp(-(z * z))

    vh = pre_v + unitb * wh16t_ref[...]
    ssq = jnp.dot(vh * vh, sumb_ref[...], preferred_element_type=jnp.float32)
    sh = jnp.sqrt(ssq + 1e-8)

    lin = (pre_s
           + jnp.dot(rbf, wrbf_ref[...], preferred_element_type=jnp.float32)
           + jnp.dot(sh, wshp_ref[...], preferred_element_type=jnp.float32)
           + bout_ref[...])
    msg_s = lin * _sigmoid(lin)
    gate = _sigmoid(jnp.dot(msg_s, wgate_ref[...],
                            preferred_element_type=jnp.float32) + bgate_ref[...])
    gatet = jnp.dot(gate, g3_ref[...], preferred_element_type=jnp.float32)
    vu = jnp.dot(vh, wub_ref[...], preferred_element_type=jnp.float32)
    out_ref[...] = msg_s
    outv_ref[...] = gatet * vu


def _edge_stage(g, pd, p):
    pm = p['msg']
    wu = pm['Wu']
    ones3 = jnp.ones((3, S), jnp.float32)
    ub = jnp.zeros((3, S), jnp.float32)
    wh16t = jnp.zeros((1, S), jnp.float32)
    sumb = jnp.zeros((S, 32), jnp.float32)
    wub = jnp.zeros((S, S), jnp.float32)
    for c in range(3):
        ub = ub.at[c, 32 * c:32 * c + H].set(1.0)
        wh16t = wh16t.at[0, 32 * c:32 * c + H].set(pm['Wh'][V])
        sumb = sumb.at[32 * c:32 * c + 32, :].set(jnp.eye(32, dtype=jnp.float32))
        wub = wub.at[32 * c:32 * c + H, 16 * c:16 * c + V].set(wu)
    wshp = jnp.zeros((32, S), jnp.float32).at[:H, :].set(pm['W_out'][S + RBF_DIM:])
    g3 = jnp.zeros((V, S), jnp.float32)
    for c in range(3):
        g3 = g3.at[:, 16 * c:16 * c + V].set(jnp.eye(V, dtype=jnp.float32))
    return pl.pallas_call(
        _edge_kernel,
        grid=(E // EB,),
        in_specs=[
            pl.BlockSpec((EB, S), lambda i: (i, 0)),
            pl.BlockSpec((8, EB), lambda i: (0, i)),
            pl.BlockSpec((3, S), lambda i: (0, 0)),
            pl.BlockSpec((3, S), lambda i: (0, 0)),
            pl.BlockSpec((1, S), lambda i: (0, 0)),
            pl.BlockSpec((S, 32), lambda i: (0, 0)),
            pl.BlockSpec((RBF_DIM, S), lambda i: (0, 0)),
            pl.BlockSpec((32, S), lambda i: (0, 0)),
            pl.BlockSpec((1, S), lambda i: (0, 0)),
            pl.BlockSpec((S, V), lambda i: (0, 0)),
            pl.BlockSpec((1, V), lambda i: (0, 0)),
            pl.BlockSpec((V, S), lambda i: (0, 0)),
            pl.BlockSpec((S, S), lambda i: (0, 0)),
        ],
        out_specs=[
            pl.BlockSpec((EB, S), lambda i: (i, 0)),
            pl.BlockSpec((EB, S), lambda i: (i, 0)),
        ],
        out_shape=[
            jax.ShapeDtypeStruct((E, S), jnp.float32),
            jax.ShapeDtypeStruct((E, S), jnp.float32),
        ],
    )(g, pd, ones3, ub, wh16t, sumb, pm['W_out'][S:S + RBF_DIM], wshp,
      pm['b_out'][None, :], pm['W_gate'], pm['b_gate'][None, :], g3, wub)


def _node_kernel(sf_ref, cf_ref, agg_ref, wh_ref, wu_ref, wouts_ref, woutv_ref,
                 bout_ref, wgate_ref, bgate_ref, s_out_ref, v_out_ref):
    agg_s = agg_ref[0] * (1.0 / NORM)
    agg_v = [agg_ref[1, :, c * V:(c + 1) * V] * (1.0 / NORM) for c in range(3)]

    mu = jnp.mean(agg_s, axis=1, keepdims=True)
    var = jnp.mean((agg_s - mu) ** 2, axis=1, keepdims=True)
    nf = (agg_s - mu) / jnp.sqrt(var + 1e-5)
    vsq = jnp.maximum(agg_v[0] ** 2 + agg_v[1] ** 2 + agg_v[2] ** 2, 1e-8)
    vn = jnp.sqrt(jnp.mean(vsq, axis=1, keepdims=True))
    inv_vn = 1.0 / vn
    s1 = sf_ref[...] + nf
    v1 = [cf_ref[:, c * V:(c + 1) * V] + agg_v[c] * inv_vn for c in range(3)]

    vh = [jnp.dot(v1[c], wh_ref[...], preferred_element_type=jnp.float32)
          for c in range(3)]
    ssq = jnp.maximum(vh[0] ** 2 + vh[1] ** 2 + vh[2] ** 2, 1e-8)
    sh = jnp.sqrt(ssq)
    lin = (jnp.dot(s1, wouts_ref[...], preferred_element_type=jnp.float32)
           + jnp.dot(sh, woutv_ref[...], preferred_element_type=jnp.float32)
           + bout_ref[...])
    f_out = lin * _sigmoid(lin)
    gate = _sigmoid(jnp.dot(f_out, wgate_ref[...],
                            preferred_element_type=jnp.float32) + bgate_ref[...])
    uv = [gate * jnp.dot(vh[c], wu_ref[...], preferred_element_type=jnp.float32)
          for c in range(3)]

    s2p = s1 + f_out
    v2p = [v1[c] + uv[c] for c in range(3)]
    mu2 = jnp.mean(s2p, axis=1, keepdims=True)
    var2 = jnp.mean((s2p - mu2) ** 2, axis=1, keepdims=True)
    s_out_ref[...] = (s2p - mu2) / jnp.sqrt(var2 + 1e-5)
    vsq2 = jnp.maximum(v2p[0] ** 2 + v2p[1] ** 2 + v2p[2] ** 2, 1e-8)
    inv_vn2 = 1.0 / jnp.sqrt(jnp.mean(vsq2, axis=1, keepdims=True))
    v_out_ref[...] = jnp.concatenate([v2p[c] * inv_vn2 for c in range(3)], axis=1)


def _node_stage(scalar_feats, cf_packed, agg, p):
    pu = p['upd']
    return pl.pallas_call(
        _node_kernel,
        grid=(N // NB,),
        in_specs=[
            pl.BlockSpec((NB, S), lambda i: (i, 0)),
            pl.BlockSpec((NB, 3 * V), lambda i: (i, 0)),
            pl.BlockSpec((2, NB, S), lambda i: (0, i, 0)),
            pl.BlockSpec((V, V), lambda i: (0, 0)),
            pl.BlockSpec((V, V), lambda i: (0, 0)),
            pl.BlockSpec((S, S), lambda i: (0, 0)),
            pl.BlockSpec((V, S), lambda i: (0, 0)),
            pl.BlockSpec((1, S), lambda i: (0, 0)),
            pl.BlockSpec((S, V), lambda i: (0, 0)),
            pl.BlockSpec((1, V), lambda i: (0, 0)),
        ],
        out_specs=[
            pl.BlockSpec((NB, S), lambda i: (i, 0)),
            pl.BlockSpec((NB, 3 * V), lambda i: (i, 0)),
        ],
        out_shape=[
            jax.ShapeDtypeStruct((N, S), jnp.float32),
            jax.ShapeDtypeStruct((N, 3 * V), jnp.float32),
        ],
    )(scalar_feats, cf_packed, agg, pu['Wh'], pu['Wu'], pu['W_out'][:S],
      pu['W_out'][S:], pu['b_out'][None, :], pu['W_gate'], pu['b_gate'][None, :])


_GCH = 128
_GNCHUNK = E // _GCH


def _sc_gather_body(tbl_hbm, pos_hbm, src_hbm, dst_hbm, out_hbm, pd_hbm,
                    idx_v, idx2_v, rows_v, pd_v, pos_v, lsem, gsem, wsem):
    cid = lax.axis_index("c")
    sid = lax.axis_index("s")
    wid = sid * _SC_NC + cid
    nfull = _GNCHUNK // _NW
    nch = nfull + jnp.where(wid < _GNCHUNK - nfull * _NW, 1, 0)

    pltpu.sync_copy(pos_hbm, pos_v)
    for b in range(2):
        for r in range(3, 8):
            for k in range(_GCH // 16):
                pd_v[b, r, pl.ds(k * 16, 16)] = jnp.zeros((16,), jnp.float32)

    def _issue_loads(i, b):
        off = (i * _NW + wid) * _GCH
        pltpu.async_copy(src_hbm.at[pl.ds(off, _GCH)], idx_v.at[b], lsem.at[b])
        pltpu.async_copy(dst_hbm.at[pl.ds(off, _GCH)], idx2_v.at[b], lsem.at[b])

    _issue_loads(0, 0)

    def _step(i, _):
        for b in range(2):
            ii = i * 2 + b
            pltpu.make_async_copy(src_hbm.at[pl.ds(0, _GCH)], idx_v.at[b], lsem.at[b]).wait()
            pltpu.make_async_copy(dst_hbm.at[pl.ds(0, _GCH)], idx2_v.at[b], lsem.at[b]).wait()

            @pl.when(ii >= 2)
            def _():
                pltpu.make_async_copy(tbl_hbm.at[pl.ds(0, _GCH)], rows_v.at[b], wsem.at[b]).wait()
                pltpu.make_async_copy(pd_hbm.at[:, pl.ds(0, _GCH)], pd_v.at[b], wsem.at[b]).wait()

            pltpu.async_copy(tbl_hbm.at[idx_v.at[b]], rows_v.at[b], gsem.at[b])

            @pl.when(ii + 1 < nch)
            def _():
                _issue_loads(ii + 1, (b + 1) % 2)

            for k in range(_GCH // 16):
                d4 = idx2_v[b, pl.ds(k * 16, 16)] * 4
                s4 = idx_v[b, pl.ds(k * 16, 16)] * 4
                for c in range(3):
                    pd_v[b, c, pl.ds(k * 16, 16)] = (
                        plsc.load_gather(pos_v, [d4 + c])
                        - plsc.load_gather(pos_v, [s4 + c]))
            pltpu.make_async_copy(tbl_hbm.at[pl.ds(0, _GCH)], rows_v.at[b], gsem.at[b]).wait()
            off = (ii * _NW + wid) * _GCH
            pltpu.async_copy(rows_v.at[b], out_hbm.at[pl.ds(off, _GCH)], wsem.at[b])
            pltpu.async_copy(pd_v.at[b], pd_hbm.at[:, pl.ds(off, _GCH)], wsem.at[b])
        return _

    lax.fori_loop(0, nch // 2, _step, None)

    @pl.when(nch % 2 == 1)
    def _():
        ii = nch - 1
        b = 0
        pltpu.make_async_copy(src_hbm.at[pl.ds(0, _GCH)], idx_v.at[b], lsem.at[b]).wait()
        pltpu.make_async_copy(dst_hbm.at[pl.ds(0, _GCH)], idx2_v.at[b], lsem.at[b]).wait()

        @pl.when(ii >= 2)
        def _():
            pltpu.make_async_copy(tbl_hbm.at[pl.ds(0, _GCH)], rows_v.at[b], wsem.at[b]).wait()
            pltpu.make_async_copy(pd_hbm.at[:, pl.ds(0, _GCH)], pd_v.at[b], wsem.at[b]).wait()

        pltpu.async_copy(tbl_hbm.at[idx_v.at[b]], rows_v.at[b], gsem.at[b])
        for k in range(_GCH // 16):
            d4 = idx2_v[b, pl.ds(k * 16, 16)] * 4
            s4 = idx_v[b, pl.ds(k * 16, 16)] * 4
            for c in range(3):
                pd_v[b, c, pl.ds(k * 16, 16)] = (
                    plsc.load_gather(pos_v, [d4 + c])
                    - plsc.load_gather(pos_v, [s4 + c]))
        pltpu.make_async_copy(tbl_hbm.at[pl.ds(0, _GCH)], rows_v.at[b], gsem.at[b]).wait()
        off = (ii * _NW + wid) * _GCH
        pltpu.async_copy(rows_v.at[b], out_hbm.at[pl.ds(off, _GCH)], wsem.at[b])
        pltpu.async_copy(pd_v.at[b], pd_hbm.at[:, pl.ds(off, _GCH)], wsem.at[b])

    @pl.when(nch >= 2)
    def _():
        pltpu.make_async_copy(tbl_hbm.at[pl.ds(0, _GCH)], rows_v.at[1], wsem.at[1]).wait()
        pltpu.make_async_copy(pd_hbm.at[:, pl.ds(0, _GCH)], pd_v.at[1], wsem.at[1]).wait()

    @pl.when(nch >= 1)
    def _():
        pltpu.make_async_copy(tbl_hbm.at[pl.ds(0, _GCH)], rows_v.at[0], wsem.at[0]).wait()
        pltpu.make_async_copy(pd_hbm.at[:, pl.ds(0, _GCH)], pd_v.at[0], wsem.at[0]).wait()


def _sc_gather(table, posflat, src, dst):
    mesh = plsc.VectorSubcoreMesh(core_axis_name="c", subcore_axis_name="s")
    f = functools.partial(
        pl.kernel,
        mesh=mesh,
        out_type=[
            jax.ShapeDtypeStruct((E, S), jnp.int32),
            jax.ShapeDtypeStruct((8, E), jnp.float32),
        ],
        scratch_types=[
            pltpu.VMEM((2, _GCH), jnp.int32),
            pltpu.VMEM((2, _GCH), jnp.int32),
            pltpu.VMEM((2, _GCH, S), jnp.int32),
            pltpu.VMEM((2, 8, _GCH), jnp.float32),
            pltpu.VMEM((4 * N,), jnp.float32),
            pltpu.SemaphoreType.DMA((2,)),
            pltpu.SemaphoreType.DMA((2,)),
            pltpu.SemaphoreType.DMA((2,)),
        ],
        compiler_params=pltpu.CompilerParams(needs_layout_passes=False),
    )(_sc_gather_body)
    return f(table, posflat, src, dst)


_SC_NC = 2
_SC_NS = 16
_NW = _SC_NC * _SC_NS
_CH = 40
_EPW = E // _NW
_EPC = E // _SC_NS


def _sc_scatter_body(msga_hbm, msgb_hbm, dst_hbm, out_hbm, idx_v, rows_v,
                     zero_v, acc_sh, lsem, asem):
    cid = lax.axis_index("c")
    sid = lax.axis_index("s")

    for r in range(16):
        for k in range(S // 16):
            zero_v[r, pl.ds(k * 16, 16)] = jnp.zeros((16,), jnp.float32)

    def _zero(j, _):
        pltpu.sync_copy(zero_v, acc_sh.at[pl.ds((j * _SC_NS + sid) * 16, 16)])
        return _

    lax.fori_loop(0, N // (16 * _SC_NS), _zero, None)
    nblk = N // 16
    done = (N // (16 * _SC_NS)) * _SC_NS

    @pl.when(sid < nblk - done)
    def _():
        pltpu.sync_copy(zero_v, acc_sh.at[pl.ds((done + sid) * 16, 16)])

    plsc.subcore_barrier()

    base = sid * _EPC
    nchunk = _EPC // _CH

    def _issue_loads(msg_hbm, c, b):
        off = base + c * _CH
        pltpu.async_copy(dst_hbm.at[pl.ds(off, _CH)], idx_v.at[b], lsem.at[b])
        pltpu.async_copy(msg_hbm.at[pl.ds(off, _CH)], rows_v.at[b], lsem.at[b])

    def _drain_loads(msg_hbm, b):
        pltpu.make_async_copy(dst_hbm.at[pl.ds(0, _CH)], idx_v.at[b], lsem.at[b]).wait()
        pltpu.make_async_copy(msg_hbm.at[pl.ds(0, _CH)], rows_v.at[b], lsem.at[b]).wait()

    def _drain_add(msg_hbm, b):
        pltpu.make_async_copy(msg_hbm.at[pl.ds(0, _CH)], rows_v.at[b], asem.at[b]).wait()

    def _run(msg_hbm):
        for b in range(2):
            _issue_loads(msg_hbm, b, b)

        def _step(i, _):
            for b in range(4):
                c = i * 4 + b
                _drain_loads(msg_hbm, b)
                pltpu.async_copy(rows_v.at[b], acc_sh.at[idx_v.at[b]],
                                 asem.at[b], add=True)
                nc = c + 2
                nb = (b + 2) % 4

                @pl.when(nc >= 4)
                def _():
                    _drain_add(msg_hbm, nb)

                @pl.when(nc < nchunk)
                def _():
                    _issue_loads(msg_hbm, nc, nb)

            return _

        lax.fori_loop(0, nchunk // 4, _step, None)
        _drain_add(msg_hbm, 2)
        _drain_add(msg_hbm, 3)

    @pl.when(cid == 0)
    def _():
        _run(msga_hbm)

    @pl.when(cid == 1)
    def _():
        _run(msgb_hbm)

    plsc.subcore_barrier()

    rows_per = 632

    @pl.when(sid < _SC_NS - 1)
    def _():
        r0 = sid * rows_per
        pltpu.sync_copy(acc_sh.at[pl.ds(r0, rows_per)],
                        out_hbm.at[cid].at[pl.ds(r0, rows_per)])

    @pl.when(sid == _SC_NS - 1)
    def _():
        r0 = (_SC_NS - 1) * rows_per
        pltpu.sync_copy(acc_sh.at[pl.ds(r0, N - r0)],
                        out_hbm.at[cid].at[pl.ds(r0, N - r0)])


def _sc_scatter(msgs_a, msgs_b, dst):
    mesh = plsc.VectorSubcoreMesh(core_axis_name="c", subcore_axis_name="s")
    f = functools.partial(
        pl.kernel,
        mesh=mesh,
        out_type=jax.ShapeDtypeStruct((_SC_NC, N, S), jnp.float32),
        scratch_types=[
            pltpu.VMEM((4, _CH), jnp.int32),
            pltpu.VMEM((4, _CH, S), jnp.float32),
            pltpu.VMEM((16, S), jnp.float32),
            pltpu.VMEM_SHARED((N, S), jnp.float32),
            pltpu.SemaphoreType.DMA((4,)),
            pltpu.SemaphoreType.DMA((4,)),
        ],
    )(_sc_scatter_body)
    return f(msgs_a, msgs_b, dst)


def kernel(scalar_feats, coord_feats, positions, edge_index, params):
    src = edge_index[0]
    dst = edge_index[1]
    cf_packed = jnp.transpose(coord_feats, (0, 2, 1)).reshape(N, 3 * V)

    table = _build_table(scalar_feats, cf_packed, positions,
                         params['msg']['W_out'][:S], params['msg']['Wh'])

    posflat = jnp.pad(positions, ((0, 0), (0, 1))).reshape(-1)
    g, pd = _sc_gather(table, posflat, src, dst)
    msgs_a, msgs_b = _edge_stage(g, pd, params)
    agg = _sc_scatter(msgs_a, msgs_b, dst)

    s2, v2p = _node_stage(scalar_feats, cf_packed, agg, params)
    v2 = jnp.transpose(v2p.reshape(N, 3, V), (0, 2, 1))
    return s2, v2


# speedup vs baseline: 1.0425x; 1.0024x over previous
import functools
import math

import jax
import jax.numpy as jnp
from jax import lax
from jax.experimental import pallas as pl
from jax.experimental.pallas import tpu as pltpu
from jax.experimental.pallas import tpu_sc as plsc

N = 10000
E = 320000
S = 128
V = 16
RBF_DIM = 16
RBF_DMAX = 15.0
NORM = 10.0
H = V + 1

TBL = 256
MSG = 176

EB = 3200
NB = 2000


def _sigmoid(x):
    return 1.0 / (1.0 + jnp.exp(-x))


def _pre_kernel(sf_ref, cf_ref, pos_ref, wouts_ref, wh_ref, out_ref):
    pre_s = jnp.dot(sf_ref[...], wouts_ref[...], preferred_element_type=jnp.float32)
    whv = wh_ref[:V, :]
    nb = sf_ref.shape[0]
    blocks = []
    for c in range(3):
        blocks.append(jnp.dot(cf_ref[:, c * V:(c + 1) * V], whv,
                              preferred_element_type=jnp.float32))
        blocks.append(jnp.zeros((nb, 32 - H), jnp.float32))
    blocks.append(jnp.zeros((nb, 32), jnp.float32))
    pre_v = jnp.concatenate(blocks, axis=1)
    au = jax.lax.bitcast_convert_type(pre_s.astype(jnp.bfloat16),
                                      jnp.uint16).astype(jnp.int32)
    bu = jax.lax.bitcast_convert_type(pre_v.astype(jnp.bfloat16),
                                      jnp.uint16).astype(jnp.int32)
    out_ref[...] = au | (bu << 16)


def _build_table(scalar_feats, cf_packed, positions, w_out_s, wh):
    return pl.pallas_call(
        _pre_kernel,
        grid=(N // NB,),
        in_specs=[
            pl.BlockSpec((NB, S), lambda i: (i, 0)),
            pl.BlockSpec((NB, 3 * V), lambda i: (i, 0)),
            pl.BlockSpec((NB, 3), lambda i: (i, 0)),
            pl.BlockSpec((S, S), lambda i: (0, 0)),
            pl.BlockSpec((H, H), lambda i: (0, 0)),
        ],
        out_specs=pl.BlockSpec((NB, S), lambda i: (i, 0)),
        out_shape=jax.ShapeDtypeStruct((N, S), jnp.int32),
    )(scalar_feats, cf_packed, positions, w_out_s, wh)


def _edge_kernel(g_ref, pd_ref, ones3_ref, ub_ref, wh16t_ref, sumb_ref,
                 wrbf_ref, wshp_ref, bout_ref, wgate_ref, bgate_ref, g3_ref,
                 wub_ref, out_ref, outv_ref):
    gi = g_ref[...]
    pre_s = jax.lax.bitcast_convert_type(gi << 16, jnp.float32)
    pre_v = jax.lax.bitcast_convert_type(gi & jnp.int32(-65536), jnp.float32)
    xd = jnp.transpose(pd_ref[...], (1, 0))[:, :3]
    d2b = jnp.dot(xd * xd, ones3_ref[...],
                  preferred_element_type=jnp.float32) + 1e-8
    inv_b = jax.lax.rsqrt(d2b)
    xdb = jnp.dot(xd, ub_ref[...], preferred_element_type=jnp.float32)
    unitb = xdb * inv_b

    j = jax.lax.broadcasted_iota(jnp.int32, (1, RBF_DIM), 1).astype(jnp.float32)
    sigma = RBF_DMAX / RBF_DIM
    dist16 = d2b[:, :RBF_DIM] * inv_b[:, :RBF_DIM]
    z = (dist16 - j * (RBF_DMAX / (RBF_DIM - 1))) / sigma
    rbf = jnp.exp(-(z * z))

    vh = pre_v + unitb * wh16t_ref[...]
    ssq = jnp.dot(vh * vh, sumb_ref[...], preferred_element_type=jnp.float32)
    sh = jnp.sqrt(ssq + 1e-8)

    lin = (pre_s
           + jnp.dot(rbf, wrbf_ref[...], preferred_element_type=jnp.float32)
           + jnp.dot(sh, wshp_ref[...], preferred_element_type=jnp.float32)
           + bout_ref[...])
    msg_s = lin * _sigmoid(lin)
    gate = _sigmoid(jnp.dot(msg_s, wgate_ref[...],
                            preferred_element_type=jnp.float32) + bgate_ref[...])
    gatet = jnp.dot(gate, g3_ref[...], preferred_element_type=jnp.float32)
    vu = jnp.dot(vh, wub_ref[...], preferred_element_type=jnp.float32)
    out_ref[...] = msg_s
    outv_ref[...] = gatet * vu


def _edge_stage(g, pd, p):
    pm = p['msg']
    wu = pm['Wu']
    ones3 = jnp.ones((3, S), jnp.float32)
    ub = jnp.zeros((3, S), jnp.float32)
    wh16t = jnp.zeros((1, S), jnp.float32)
    sumb = jnp.zeros((S, 32), jnp.float32)
    wub = jnp.zeros((S, S), jnp.float32)
    for c in range(3):
        ub = ub.at[c, 32 * c:32 * c + H].set(1.0)
        wh16t = wh16t.at[0, 32 * c:32 * c + H].set(pm['Wh'][V])
        sumb = sumb.at[32 * c:32 * c + 32, :].set(jnp.eye(32, dtype=jnp.float32))
        wub = wub.at[32 * c:32 * c + H, 16 * c:16 * c + V].set(wu)
    wshp = jnp.zeros((32, S), jnp.float32).at[:H, :].set(pm['W_out'][S + RBF_DIM:])
    g3 = jnp.zeros((V, S), jnp.float32)
    for c in range(3):
        g3 = g3.at[:, 16 * c:16 * c + V].set(jnp.eye(V, dtype=jnp.float32))
    return pl.pallas_call(
        _edge_kernel,
        grid=(E // EB,),
        in_specs=[
            pl.BlockSpec((EB, S), lambda i: (i, 0)),
            pl.BlockSpec((8, EB), lambda i: (0, i)),
            pl.BlockSpec((3, S), lambda i: (0, 0)),
            pl.BlockSpec((3, S), lambda i: (0, 0)),
            pl.BlockSpec((1, S), lambda i: (0, 0)),
            pl.BlockSpec((S, 32), lambda i: (0, 0)),
            pl.BlockSpec((RBF_DIM, S), lambda i: (0, 0)),
            pl.BlockSpec((32, S), lambda i: (0, 0)),
            pl.BlockSpec((1, S), lambda i: (0, 0)),
            pl.BlockSpec((S, V), lambda i: (0, 0)),
            pl.BlockSpec((1, V), lambda i: (0, 0)),
            pl.BlockSpec((V, S), lambda i: (0, 0)),
            pl.BlockSpec((S, S), lambda i: (0, 0)),
        ],
        out_specs=[
            pl.BlockSpec((EB, S), lambda i: (i, 0)),
            pl.BlockSpec((EB, S), lambda i: (i, 0)),
        ],
        out_shape=[
            jax.ShapeDtypeStruct((E, S), jnp.float32),
            jax.ShapeDtypeStruct((E, S), jnp.float32),
        ],
    )(g, pd, ones3, ub, wh16t, sumb, pm['W_out'][S:S + RBF_DIM], wshp,
      pm['b_out'][None, :], pm['W_gate'], pm['b_gate'][None, :], g3, wub)


def _node_kernel(sf_ref, cf_ref, agg_ref, wh_ref, wu_ref, wouts_ref, woutv_ref,
                 bout_ref, wgate_ref, bgate_ref, s_out_ref, v_out_ref):
    agg_s = agg_ref[0] * (1.0 / NORM)
    agg_v = [agg_ref[1, :, c * V:(c + 1) * V] * (1.0 / NORM) for c in range(3)]

    mu = jnp.mean(agg_s, axis=1, keepdims=True)
    var = jnp.mean((agg_s - mu) ** 2, axis=1, keepdims=True)
    nf = (agg_s - mu) / jnp.sqrt(var + 1e-5)
    vsq = jnp.maximum(agg_v[0] ** 2 + agg_v[1] ** 2 + agg_v[2] ** 2, 1e-8)
    vn = jnp.sqrt(jnp.mean(vsq, axis=1, keepdims=True))
    inv_vn = 1.0 / vn
    s1 = sf_ref[...] + nf
    v1 = [cf_ref[:, c * V:(c + 1) * V] + agg_v[c] * inv_vn for c in range(3)]

    vh = [jnp.dot(v1[c], wh_ref[...], preferred_element_type=jnp.float32)
          for c in range(3)]
    ssq = jnp.maximum(vh[0] ** 2 + vh[1] ** 2 + vh[2] ** 2, 1e-8)
    sh = jnp.sqrt(ssq)
    lin = (jnp.dot(s1, wouts_ref[...], preferred_element_type=jnp.float32)
           + jnp.dot(sh, woutv_ref[...], preferred_element_type=jnp.float32)
           + bout_ref[...])
    f_out = lin * _sigmoid(lin)
    gate = _sigmoid(jnp.dot(f_out, wgate_ref[...],
                            preferred_element_type=jnp.float32) + bgate_ref[...])
    uv = [gate * jnp.dot(vh[c], wu_ref[...], preferred_element_type=jnp.float32)
          for c in range(3)]

    s2p = s1 + f_out
    v2p = [v1[c] + uv[c] for c in range(3)]
    mu2 = jnp.mean(s2p, axis=1, keepdims=True)
    var2 = jnp.mean((s2p - mu2) ** 2, axis=1, keepdims=True)
    s_out_ref[...] = (s2p - mu2) / jnp.sqrt(var2 + 1e-5)
    vsq2 = jnp.maximum(v2p[0] ** 2 + v2p[1] ** 2 + v2p[2] ** 2, 1e-8)
    inv_vn2 = 1.0 / jnp.sqrt(jnp.mean(vsq2, axis=1, keepdims=True))
    v_out_ref[...] = jnp.concatenate([v2p[c] * inv_vn2 for c in range(3)], axis=1)


def _node_stage(scalar_feats, cf_packed, agg, p):
    pu = p['upd']
    return pl.pallas_call(
        _node_kernel,
        grid=(N // NB,),
        in_specs=[
            pl.BlockSpec((NB, S), lambda i: (i, 0)),
            pl.BlockSpec((NB, 3 * V), lambda i: (i, 0)),
            pl.BlockSpec((2, NB, S), lambda i: (0, i, 0)),
            pl.BlockSpec((V, V), lambda i: (0, 0)),
            pl.BlockSpec((V, V), lambda i: (0, 0)),
            pl.BlockSpec((S, S), lambda i: (0, 0)),
            pl.BlockSpec((V, S), lambda i: (0, 0)),
            pl.BlockSpec((1, S), lambda i: (0, 0)),
            pl.BlockSpec((S, V), lambda i: (0, 0)),
            pl.BlockSpec((1, V), lambda i: (0, 0)),
        ],
        out_specs=[
            pl.BlockSpec((NB, S), lambda i: (i, 0)),
            pl.BlockSpec((NB, 3 * V), lambda i: (i, 0)),
        ],
        out_shape=[
            jax.ShapeDtypeStruct((N, S), jnp.float32),
            jax.ShapeDtypeStruct((N, 3 * V), jnp.float32),
        ],
    )(scalar_feats, cf_packed, agg, pu['Wh'], pu['Wu'], pu['W_out'][:S],
      pu['W_out'][S:], pu['b_out'][None, :], pu['W_gate'], pu['b_gate'][None, :])


_GCH = 128
_GNCHUNK = E // _GCH


def _sc_gather_body(tbl_hbm, pos_hbm, src_hbm, dst_hbm, out_hbm, pd_hbm,
                    idx_v, idx2_v, rows_v, pd_v, pos_v, lsem, gsem, wsem):
    cid = lax.axis_index("c")
    sid = lax.axis_index("s")
    wid = sid * _SC_NC + cid
    nfull = _GNCHUNK // _NW
    nch = nfull + jnp.where(wid < _GNCHUNK - nfull * _NW, 1, 0)

    pltpu.sync_copy(pos_hbm, pos_v)
    for b in range(2):
        for r in range(3, 8):
            for k in range(_GCH // 16):
                pd_v[b, r, pl.ds(k * 16, 16)] = jnp.zeros((16,), jnp.float32)

    def _issue_loads(i, b):
        off = (i * _NW + wid) * _GCH
        pltpu.async_copy(src_hbm.at[pl.ds(off, _GCH)], idx_v.at[b], lsem.at[b])
        pltpu.async_copy(dst_hbm.at[pl.ds(off, _GCH)], idx2_v.at[b], lsem.at[b])

    _issue_loads(0, 0)

    def _step(i, _):
        for b in range(2):
            ii = i * 2 + b
            pltpu.make_async_copy(src_hbm.at[pl.ds(0, _GCH)], idx_v.at[b], lsem.at[b]).wait()
            pltpu.make_async_copy(dst_hbm.at[pl.ds(0, _GCH)], idx2_v.at[b], lsem.at[b]).wait()

            @pl.when(ii >= 2)
            def _():
                pltpu.make_async_copy(tbl_hbm.at[pl.ds(0, _GCH)], rows_v.at[b], wsem.at[b]).wait()
                pltpu.make_async_copy(pd_hbm.at[:, pl.ds(0, _GCH)], pd_v.at[b], wsem.at[b]).wait()

            pltpu.async_copy(tbl_hbm.at[idx_v.at[b]], rows_v.at[b], gsem.at[b])

            @pl.when(ii + 1 < nch)
            def _():
                _issue_loads(ii + 1, (b + 1) % 2)

            for k in range(_GCH // 16):
                d4 = idx2_v[b, pl.ds(k * 16, 16)] * 4
                s4 = idx_v[b, pl.ds(k * 16, 16)] * 4
                for c in range(3):
                    pd_v[b, c, pl.ds(k * 16, 16)] = (
                        plsc.load_gather(pos_v, [d4 + c])
                        - plsc.load_gather(pos_v, [s4 + c]))
            pltpu.make_async_copy(tbl_hbm.at[pl.ds(0, _GCH)], rows_v.at[b], gsem.at[b]).wait()
            off = (ii * _NW + wid) * _GCH
            pltpu.async_copy(rows_v.at[b], out_hbm.at[pl.ds(off, _GCH)], wsem.at[b])
            pltpu.async_copy(pd_v.at[b], pd_hbm.at[:, pl.ds(off, _GCH)], wsem.at[b])
        return _

    lax.fori_loop(0, nch // 2, _step, None)

    @pl.when(nch % 2 == 1)
    def _():
        ii = nch - 1
        b = 0
        pltpu.make_async_copy(src_hbm.at[pl.ds(0, _GCH)], idx_v.at[b], lsem.at[b]).wait()
        pltpu.make_async_copy(dst_hbm.at[pl.ds(0, _GCH)], idx2_v.at[b], lsem.at[b]).wait()

        @pl.when(ii >= 2)
        def _():
            pltpu.make_async_copy(tbl_hbm.at[pl.ds(0, _GCH)], rows_v.at[b], wsem.at[b]).wait()
            pltpu.make_async_copy(pd_hbm.at[:, pl.ds(0, _GCH)], pd_v.at[b], wsem.at[b]).wait()

        pltpu.async_copy(tbl_hbm.at[idx_v.at[b]], rows_v.at[b], gsem.at[b])
        for k in range(_GCH // 16):
            d4 = idx2_v[b, pl.ds(k * 16, 16)] * 4
            s4 = idx_v[b, pl.ds(k * 16, 16)] * 4
            for c in range(3):
                pd_v[b, c, pl.ds(k * 16, 16)] = (
                    plsc.load_gather(pos_v, [d4 + c])
                    - plsc.load_gather(pos_v, [s4 + c]))
        pltpu.make_async_copy(tbl_hbm.at[pl.ds(0, _GCH)], rows_v.at[b], gsem.at[b]).wait()
        off = (ii * _NW + wid) * _GCH
        pltpu.async_copy(rows_v.at[b], out_hbm.at[pl.ds(off, _GCH)], wsem.at[b])
        pltpu.async_copy(pd_v.at[b], pd_hbm.at[:, pl.ds(off, _GCH)], wsem.at[b])

    @pl.when(nch >= 2)
    def _():
        pltpu.make_async_copy(tbl_hbm.at[pl.ds(0, _GCH)], rows_v.at[1], wsem.at[1]).wait()
        pltpu.make_async_copy(pd_hbm.at[:, pl.ds(0, _GCH)], pd_v.at[1], wsem.at[1]).wait()

    @pl.when(nch >= 1)
    def _():
        pltpu.make_async_copy(tbl_hbm.at[pl.ds(0, _GCH)], rows_v.at[0], wsem.at[0]).wait()
        pltpu.make_async_copy(pd_hbm.at[:, pl.ds(0, _GCH)], pd_v.at[0], wsem.at[0]).wait()


def _sc_gather(table, posflat, src, dst):
    mesh = plsc.VectorSubcoreMesh(core_axis_name="c", subcore_axis_name="s")
    f = functools.partial(
        pl.kernel,
        mesh=mesh,
        out_type=[
            jax.ShapeDtypeStruct((E, S), jnp.int32),
            jax.ShapeDtypeStruct((8, E), jnp.float32),
        ],
        scratch_types=[
            pltpu.VMEM((2, _GCH), jnp.int32),
            pltpu.VMEM((2, _GCH), jnp.int32),
            pltpu.VMEM((2, _GCH, S), jnp.int32),
            pltpu.VMEM((2, 8, _GCH), jnp.float32),
            pltpu.VMEM((4 * N,), jnp.float32),
            pltpu.SemaphoreType.DMA((2,)),
            pltpu.SemaphoreType.DMA((2,)),
            pltpu.SemaphoreType.DMA((2,)),
        ],
        compiler_params=pltpu.CompilerParams(needs_layout_passes=False),
    )(_sc_gather_body)
    return f(table, posflat, src, dst)


_SC_NC = 2
_SC_NS = 16
_NW = _SC_NC * _SC_NS
_CH = 40
_EPW = E // _NW
_EPC = E // _SC_NS


def _sc_scatter_body(msga_hbm, msgb_hbm, dst_hbm, out_hbm, idx_v, rows_v,
                     zero_v, acc_sh, lsem, asem):
    cid = lax.axis_index("c")
    sid = lax.axis_index("s")

    for r in range(16):
        for k in range(S // 16):
            zero_v[r, pl.ds(k * 16, 16)] = jnp.zeros((16,), jnp.float32)

    def _zero(j, _):
        pltpu.sync_copy(zero_v, acc_sh.at[pl.ds((j * _SC_NS + sid) * 16, 16)])
        return _

    lax.fori_loop(0, N // (16 * _SC_NS), _zero, None)
    nblk = N // 16
    done = (N // (16 * _SC_NS)) * _SC_NS

    @pl.when(sid < nblk - done)
    def _():
        pltpu.sync_copy(zero_v, acc_sh.at[pl.ds((done + sid) * 16, 16)])

    plsc.subcore_barrier()

    base = sid * _EPC
    nchunk = _EPC // _CH

    def _issue_loads(msg_hbm, c, b):
        off = base + c * _CH
        pltpu.async_copy(dst_hbm.at[pl.ds(off, _CH)], idx_v.at[b], lsem.at[b])
        pltpu.async_copy(msg_hbm.at[pl.ds(off, _CH)], rows_v.at[b], lsem.at[b])

    def _drain_loads(msg_hbm, b):
        pltpu.make_async_copy(dst_hbm.at[pl.ds(0, _CH)], idx_v.at[b], lsem.at[b]).wait()
        pltpu.make_async_copy(msg_hbm.at[pl.ds(0, _CH)], rows_v.at[b], lsem.at[b]).wait()

    def _drain_add(msg_hbm, b):
        pltpu.make_async_copy(msg_hbm.at[pl.ds(0, _CH)], rows_v.at[b], asem.at[b]).wait()

    def _run(msg_hbm):
        for b in range(2):
            _issue_loads(msg_hbm, b, b)

        def _step(i, _):
            for b in range(4):
                c = i * 4 + b
                _drain_loads(msg_hbm, b)
                pltpu.async_copy(rows_v.at[b], acc_sh.at[idx_v.at[b]],
                                 asem.at[b], add=True)
                nc = c + 2
                nb = (b + 2) % 4

                @pl.when(nc >= 4)
                def _():
                    _drain_add(msg_hbm, nb)

                @pl.when(nc < nchunk)
                def _():
                    _issue_loads(msg_hbm, nc, nb)

            return _

        lax.fori_loop(0, nchunk // 4, _step, None)
        _drain_add(msg_hbm, 2)
        _drain_add(msg_hbm, 3)

    @pl.when(cid == 0)
    def _():
        _run(msga_hbm)

    @pl.when(cid == 1)
    def _():
        _run(msgb_hbm)

    plsc.subcore_barrier()

    rows_per = 632

    @pl.when(sid < _SC_NS - 1)
    def _():
        r0 = sid * rows_per
        pltpu.sync_copy(acc_sh.at[pl.ds(r0, rows_per)],
                        out_hbm.at[cid].at[pl.ds(r0, rows_per)])

    @pl.when(sid == _SC_NS - 1)
    def _():
        r0 = (_SC_NS - 1) * rows_per
        pltpu.sync_copy(acc_sh.at[pl.ds(r0, N - r0)],
                        out_hbm.at[cid].at[pl.ds(r0, N - r0)])


def _sc_scatter(msgs_a, msgs_b, dst):
    mesh = plsc.VectorSubcoreMesh(core_axis_name="c", subcore_axis_name="s")
    f = functools.partial(
        pl.kernel,
        mesh=mesh,
        out_type=jax.ShapeDtypeStruct((_SC_NC, N, S), jnp.float32),
        scratch_types=[
            pltpu.VMEM((4, _CH), jnp.int32),
            pltpu.VMEM((4, _CH, S), jnp.float32),
            pltpu.VMEM((16, S), jnp.float32),
            pltpu.VMEM_SHARED((N, S), jnp.float32),
            pltpu.SemaphoreType.DMA((4,)),
            pltpu.SemaphoreType.DMA((4,)),
        ],
    )(_sc_scatter_body)
    return f(msgs_a, msgs_b, dst)


def kernel(scalar_feats, coord_feats, positions, edge_index, params):
    src = edge_index[0]
    dst = edge_index[1]
    cf_packed = jnp.transpose(coord_feats, (0, 2, 1)).reshape(N, 3 * V)

    table = _build_table(scalar_feats, cf_packed, positions,
                         params['msg']['W_out'][:S], params['msg']['Wh'])

    posflat = jnp.pad(positions, ((0, 0), (0, 1))).reshape(-1)
    g, pd = _sc_gather(table, posflat, src, dst)
    msgs_a, msgs_b = _edge_stage(g, pd, params)
    agg = _sc_scatter(msgs_a, msgs_b, dst)

    s2, v2p = _node_stage(scalar_feats, cf_packed, agg, params)
    v2 = jnp.transpose(v2p.reshape(N, 3, V), (0, 2, 1))
    return s2, v2


# speedup vs baseline: 1.0483x; 1.0056x over previous
import functools

import jax
import jax.numpy as jnp
from jax import lax
from jax.experimental import pallas as pl
from jax.experimental.pallas import tpu as pltpu
from jax.experimental.pallas import tpu_sc as plsc

N = 10000
E = 320000
S = 128
V = 16
RBF_DIM = 16
RBF_DMAX = 15.0
NORM = 10.0
H = V + 1


EB = 3200
NB = 2000


def _sigmoid(x):
    return 1.0 / (1.0 + jnp.exp(-x))


def _pre_kernel(sf_ref, cf_ref, wouts_ref, wh_ref, out_ref):
    pre_s = jnp.dot(sf_ref[...], wouts_ref[...], preferred_element_type=jnp.float32)
    whv = wh_ref[:V, :]
    nb = sf_ref.shape[0]
    blocks = []
    for c in range(3):
        blocks.append(jnp.dot(cf_ref[:, c * V:(c + 1) * V], whv,
                              preferred_element_type=jnp.float32))
        blocks.append(jnp.zeros((nb, 32 - H), jnp.float32))
    blocks.append(jnp.zeros((nb, 32), jnp.float32))
    pre_v = jnp.concatenate(blocks, axis=1)
    au = jax.lax.bitcast_convert_type(pre_s.astype(jnp.bfloat16),
                                      jnp.uint16).astype(jnp.int32)
    bu = jax.lax.bitcast_convert_type(pre_v.astype(jnp.bfloat16),
                                      jnp.uint16).astype(jnp.int32)
    out_ref[...] = au | (bu << 16)


def _build_table(scalar_feats, cf_packed, w_out_s, wh):
    return pl.pallas_call(
        _pre_kernel,
        grid=(N // NB,),
        in_specs=[
            pl.BlockSpec((NB, S), lambda i: (i, 0)),
            pl.BlockSpec((NB, 3 * V), lambda i: (i, 0)),
            pl.BlockSpec((S, S), lambda i: (0, 0)),
            pl.BlockSpec((H, H), lambda i: (0, 0)),
        ],
        out_specs=pl.BlockSpec((NB, S), lambda i: (i, 0)),
        out_shape=jax.ShapeDtypeStruct((N, S), jnp.int32),
    )(scalar_feats, cf_packed, w_out_s, wh)


def _edge_kernel(g_ref, pd_ref, ones3_ref, ub_ref, wh16t_ref, sumb_ref,
                 wrbf_ref, wshp_ref, bout_ref, wgate_ref, bgate_ref, g3_ref,
                 wub_ref, out_ref, outv_ref):
    gi = g_ref[...]
    pre_s = jax.lax.bitcast_convert_type(gi << 16, jnp.float32)
    pre_v = jax.lax.bitcast_convert_type(gi & jnp.int32(-65536), jnp.float32)
    xd = jnp.transpose(pd_ref[...], (1, 0))[:, :3]
    d2b = jnp.dot(xd * xd, ones3_ref[...],
                  preferred_element_type=jnp.float32) + 1e-8
    inv_b = jax.lax.rsqrt(d2b)
    xdb = jnp.dot(xd, ub_ref[...], preferred_element_type=jnp.float32)
    unitb = xdb * inv_b

    j = jax.lax.broadcasted_iota(jnp.int32, (1, RBF_DIM), 1).astype(jnp.float32)
    sigma = RBF_DMAX / RBF_DIM
    dist16 = d2b[:, :RBF_DIM] * inv_b[:, :RBF_DIM]
    z = (dist16 - j * (RBF_DMAX / (RBF_DIM - 1))) / sigma
    rbf = jnp.exp(-(z * z))

    vh = pre_v + unitb * wh16t_ref[...]
    ssq = jnp.dot(vh * vh, sumb_ref[...], preferred_element_type=jnp.float32)
    sh = jnp.sqrt(ssq + 1e-8)

    lin = (pre_s
           + jnp.dot(rbf, wrbf_ref[...], preferred_element_type=jnp.float32)
           + jnp.dot(sh, wshp_ref[...], preferred_element_type=jnp.float32)
           + bout_ref[...])
    msg_s = lin * _sigmoid(lin)
    gate = _sigmoid(jnp.dot(msg_s, wgate_ref[...],
                            preferred_element_type=jnp.float32) + bgate_ref[...])
    gatet = jnp.dot(gate, g3_ref[...], preferred_element_type=jnp.float32)
    vu = jnp.dot(vh, wub_ref[...], preferred_element_type=jnp.float32)
    out_ref[...] = msg_s
    outv_ref[...] = gatet * vu


def _edge_stage(g, pd, p):
    pm = p['msg']
    wu = pm['Wu']
    ones3 = jnp.ones((3, S), jnp.float32)
    ub = jnp.zeros((3, S), jnp.float32)
    wh16t = jnp.zeros((1, S), jnp.float32)
    sumb = jnp.zeros((S, 32), jnp.float32)
    wub = jnp.zeros((S, S), jnp.float32)
    for c in range(3):
        ub = ub.at[c, 32 * c:32 * c + H].set(1.0)
        wh16t = wh16t.at[0, 32 * c:32 * c + H].set(pm['Wh'][V])
        sumb = sumb.at[32 * c:32 * c + 32, :].set(jnp.eye(32, dtype=jnp.float32))
        wub = wub.at[32 * c:32 * c + H, 16 * c:16 * c + V].set(wu)
    wshp = jnp.zeros((32, S), jnp.float32).at[:H, :].set(pm['W_out'][S + RBF_DIM:])
    g3 = jnp.zeros((V, S), jnp.float32)
    for c in range(3):
        g3 = g3.at[:, 16 * c:16 * c + V].set(jnp.eye(V, dtype=jnp.float32))
    return pl.pallas_call(
        _edge_kernel,
        grid=(E // EB,),
        in_specs=[
            pl.BlockSpec((EB, S), lambda i: (i, 0)),
            pl.BlockSpec((8, EB), lambda i: (0, i)),
            pl.BlockSpec((3, S), lambda i: (0, 0)),
            pl.BlockSpec((3, S), lambda i: (0, 0)),
            pl.BlockSpec((1, S), lambda i: (0, 0)),
            pl.BlockSpec((S, 32), lambda i: (0, 0)),
            pl.BlockSpec((RBF_DIM, S), lambda i: (0, 0)),
            pl.BlockSpec((32, S), lambda i: (0, 0)),
            pl.BlockSpec((1, S), lambda i: (0, 0)),
            pl.BlockSpec((S, V), lambda i: (0, 0)),
            pl.BlockSpec((1, V), lambda i: (0, 0)),
            pl.BlockSpec((V, S), lambda i: (0, 0)),
            pl.BlockSpec((S, S), lambda i: (0, 0)),
        ],
        out_specs=[
            pl.BlockSpec((EB, S), lambda i: (i, 0)),
            pl.BlockSpec((EB, S), lambda i: (i, 0)),
        ],
        out_shape=[
            jax.ShapeDtypeStruct((E, S), jnp.float32),
            jax.ShapeDtypeStruct((E, S), jnp.float32),
        ],
    )(g, pd, ones3, ub, wh16t, sumb, pm['W_out'][S:S + RBF_DIM], wshp,
      pm['b_out'][None, :], pm['W_gate'], pm['b_gate'][None, :], g3, wub)


def _node_kernel(sf_ref, cf_ref, agg_ref, wh_ref, wu_ref, wouts_ref, woutv_ref,
                 bout_ref, wgate_ref, bgate_ref, s_out_ref, v_out_ref):
    agg_s = agg_ref[0] * (1.0 / NORM)
    agg_v = [agg_ref[1, :, c * V:(c + 1) * V] * (1.0 / NORM) for c in range(3)]

    mu = jnp.mean(agg_s, axis=1, keepdims=True)
    var = jnp.mean((agg_s - mu) ** 2, axis=1, keepdims=True)
    nf = (agg_s - mu) / jnp.sqrt(var + 1e-5)
    vsq = jnp.maximum(agg_v[0] ** 2 + agg_v[1] ** 2 + agg_v[2] ** 2, 1e-8)
    vn = jnp.sqrt(jnp.mean(vsq, axis=1, keepdims=True))
    inv_vn = 1.0 / vn
    s1 = sf_ref[...] + nf
    v1 = [cf_ref[:, c * V:(c + 1) * V] + agg_v[c] * inv_vn for c in range(3)]

    vh = [jnp.dot(v1[c], wh_ref[...], preferred_element_type=jnp.float32)
          for c in range(3)]
    ssq = jnp.maximum(vh[0] ** 2 + vh[1] ** 2 + vh[2] ** 2, 1e-8)
    sh = jnp.sqrt(ssq)
    lin = (jnp.dot(s1, wouts_ref[...], preferred_element_type=jnp.float32)
           + jnp.dot(sh, woutv_ref[...], preferred_element_type=jnp.float32)
           + bout_ref[...])
    f_out = lin * _sigmoid(lin)
    gate = _sigmoid(jnp.dot(f_out, wgate_ref[...],
                            preferred_element_type=jnp.float32) + bgate_ref[...])
    uv = [gate * jnp.dot(vh[c], wu_ref[...], preferred_element_type=jnp.float32)
          for c in range(3)]

    s2p = s1 + f_out
    v2p = [v1[c] + uv[c] for c in range(3)]
    mu2 = jnp.mean(s2p, axis=1, keepdims=True)
    var2 = jnp.mean((s2p - mu2) ** 2, axis=1, keepdims=True)
    s_out_ref[...] = (s2p - mu2) / jnp.sqrt(var2 + 1e-5)
    vsq2 = jnp.maximum(v2p[0] ** 2 + v2p[1] ** 2 + v2p[2] ** 2, 1e-8)
    inv_vn2 = 1.0 / jnp.sqrt(jnp.mean(vsq2, axis=1, keepdims=True))
    v_out_ref[...] = jnp.concatenate([v2p[c] * inv_vn2 for c in range(3)], axis=1)


def _node_stage(scalar_feats, cf_packed, agg, p):
    pu = p['upd']
    return pl.pallas_call(
        _node_kernel,
        grid=(N // NB,),
        in_specs=[
            pl.BlockSpec((NB, S), lambda i: (i, 0)),
            pl.BlockSpec((NB, 3 * V), lambda i: (i, 0)),
            pl.BlockSpec((2, NB, S), lambda i: (0, i, 0)),
            pl.BlockSpec((V, V), lambda i: (0, 0)),
            pl.BlockSpec((V, V), lambda i: (0, 0)),
            pl.BlockSpec((S, S), lambda i: (0, 0)),
            pl.BlockSpec((V, S), lambda i: (0, 0)),
            pl.BlockSpec((1, S), lambda i: (0, 0)),
            pl.BlockSpec((S, V), lambda i: (0, 0)),
            pl.BlockSpec((1, V), lambda i: (0, 0)),
        ],
        out_specs=[
            pl.BlockSpec((NB, S), lambda i: (i, 0)),
            pl.BlockSpec((NB, 3 * V), lambda i: (i, 0)),
        ],
        out_shape=[
            jax.ShapeDtypeStruct((N, S), jnp.float32),
            jax.ShapeDtypeStruct((N, 3 * V), jnp.float32),
        ],
    )(scalar_feats, cf_packed, agg, pu['Wh'], pu['Wu'], pu['W_out'][:S],
      pu['W_out'][S:], pu['b_out'][None, :], pu['W_gate'], pu['b_gate'][None, :])


_GCH = 128
_GNCHUNK = E // _GCH


def _sc_gather_body(tbl_hbm, pos_hbm, src_hbm, dst_hbm, out_hbm, pd_hbm,
                    idx_v, idx2_v, rows_v, pd_v, pos_v, lsem, gsem, wsem):
    cid = lax.axis_index("c")
    sid = lax.axis_index("s")
    wid = sid * _SC_NC + cid
    nfull = _GNCHUNK // _NW
    nch = nfull + jnp.where(wid < _GNCHUNK - nfull * _NW, 1, 0)

    pltpu.sync_copy(pos_hbm, pos_v)
    for b in range(2):
        for r in range(3, 8):
            for k in range(_GCH // 16):
                pd_v[b, r, pl.ds(k * 16, 16)] = jnp.zeros((16,), jnp.float32)

    def _issue_loads(i, b):
        off = (i * _NW + wid) * _GCH
        pltpu.async_copy(src_hbm.at[pl.ds(off, _GCH)], idx_v.at[b], lsem.at[b])
        pltpu.async_copy(dst_hbm.at[pl.ds(off, _GCH)], idx2_v.at[b], lsem.at[b])

    _issue_loads(0, 0)

    def _step(i, _):
        for b in range(2):
            ii = i * 2 + b
            pltpu.make_async_copy(src_hbm.at[pl.ds(0, _GCH)], idx_v.at[b], lsem.at[b]).wait()
            pltpu.make_async_copy(dst_hbm.at[pl.ds(0, _GCH)], idx2_v.at[b], lsem.at[b]).wait()

            @pl.when(ii >= 2)
            def _():
                pltpu.make_async_copy(tbl_hbm.at[pl.ds(0, _GCH)], rows_v.at[b], wsem.at[b]).wait()
                pltpu.make_async_copy(pd_hbm.at[:, pl.ds(0, _GCH)], pd_v.at[b], wsem.at[b]).wait()

            pltpu.async_copy(tbl_hbm.at[idx_v.at[b]], rows_v.at[b], gsem.at[b])

            @pl.when(ii + 1 < nch)
            def _():
                _issue_loads(ii + 1, (b + 1) % 2)

            for k in range(_GCH // 16):
                d4 = idx2_v[b, pl.ds(k * 16, 16)] * 4
                s4 = idx_v[b, pl.ds(k * 16, 16)] * 4
                for c in range(3):
                    pd_v[b, c, pl.ds(k * 16, 16)] = (
                        plsc.load_gather(pos_v, [d4 + c])
                        - plsc.load_gather(pos_v, [s4 + c]))
            pltpu.make_async_copy(tbl_hbm.at[pl.ds(0, _GCH)], rows_v.at[b], gsem.at[b]).wait()
            off = (ii * _NW + wid) * _GCH
            pltpu.async_copy(rows_v.at[b], out_hbm.at[pl.ds(off, _GCH)], wsem.at[b])
            pltpu.async_copy(pd_v.at[b], pd_hbm.at[:, pl.ds(off, _GCH)], wsem.at[b])
        return _

    lax.fori_loop(0, nch // 2, _step, None)

    @pl.when(nch % 2 == 1)
    def _():
        ii = nch - 1
        b = 0
        pltpu.make_async_copy(src_hbm.at[pl.ds(0, _GCH)], idx_v.at[b], lsem.at[b]).wait()
        pltpu.make_async_copy(dst_hbm.at[pl.ds(0, _GCH)], idx2_v.at[b], lsem.at[b]).wait()

        @pl.when(ii >= 2)
        def _():
            pltpu.make_async_copy(tbl_hbm.at[pl.ds(0, _GCH)], rows_v.at[b], wsem.at[b]).wait()
            pltpu.make_async_copy(pd_hbm.at[:, pl.ds(0, _GCH)], pd_v.at[b], wsem.at[b]).wait()

        pltpu.async_copy(tbl_hbm.at[idx_v.at[b]], rows_v.at[b], gsem.at[b])
        for k in range(_GCH // 16):
            d4 = idx2_v[b, pl.ds(k * 16, 16)] * 4
            s4 = idx_v[b, pl.ds(k * 16, 16)] * 4
            for c in range(3):
                pd_v[b, c, pl.ds(k * 16, 16)] = (
                    plsc.load_gather(pos_v, [d4 + c])
                    - plsc.load_gather(pos_v, [s4 + c]))
        pltpu.make_async_copy(tbl_hbm.at[pl.ds(0, _GCH)], rows_v.at[b], gsem.at[b]).wait()
        off = (ii * _NW + wid) * _GCH
        pltpu.async_copy(rows_v.at[b], out_hbm.at[pl.ds(off, _GCH)], wsem.at[b])
        pltpu.async_copy(pd_v.at[b], pd_hbm.at[:, pl.ds(off, _GCH)], wsem.at[b])

    @pl.when(nch >= 2)
    def _():
        pltpu.make_async_copy(tbl_hbm.at[pl.ds(0, _GCH)], rows_v.at[1], wsem.at[1]).wait()
        pltpu.make_async_copy(pd_hbm.at[:, pl.ds(0, _GCH)], pd_v.at[1], wsem.at[1]).wait()

    @pl.when(nch >= 1)
    def _():
        pltpu.make_async_copy(tbl_hbm.at[pl.ds(0, _GCH)], rows_v.at[0], wsem.at[0]).wait()
        pltpu.make_async_copy(pd_hbm.at[:, pl.ds(0, _GCH)], pd_v.at[0], wsem.at[0]).wait()


def _sc_gather(table, posflat, src, dst):
    mesh = plsc.VectorSubcoreMesh(core_axis_name="c", subcore_axis_name="s")
    f = functools.partial(
        pl.kernel,
        mesh=mesh,
        out_type=[
            jax.ShapeDtypeStruct((E, S), jnp.int32),
            jax.ShapeDtypeStruct((8, E), jnp.float32),
        ],
        scratch_types=[
            pltpu.VMEM((2, _GCH), jnp.int32),
            pltpu.VMEM((2, _GCH), jnp.int32),
            pltpu.VMEM((2, _GCH, S), jnp.int32),
            pltpu.VMEM((2, 8, _GCH), jnp.float32),
            pltpu.VMEM((4 * N,), jnp.float32),
            pltpu.SemaphoreType.DMA((2,)),
            pltpu.SemaphoreType.DMA((2,)),
            pltpu.SemaphoreType.DMA((2,)),
        ],
        compiler_params=pltpu.CompilerParams(needs_layout_passes=False),
    )(_sc_gather_body)
    return f(table, posflat, src, dst)


_SC_NC = 2
_SC_NS = 16
_NW = _SC_NC * _SC_NS
_CH = 40
_EPC = E // _SC_NS


def _sc_scatter_body(msga_hbm, msgb_hbm, dst_hbm, out_hbm, idx_v, rows_v,
                     zero_v, acc_sh, lsem, asem):
    cid = lax.axis_index("c")
    sid = lax.axis_index("s")

    for r in range(16):
        for k in range(S // 16):
            zero_v[r, pl.ds(k * 16, 16)] = jnp.zeros((16,), jnp.float32)

    def _zero(j, _):
        pltpu.sync_copy(zero_v, acc_sh.at[pl.ds((j * _SC_NS + sid) * 16, 16)])
        return _

    lax.fori_loop(0, N // (16 * _SC_NS), _zero, None)
    nblk = N // 16
    done = (N // (16 * _SC_NS)) * _SC_NS

    @pl.when(sid < nblk - done)
    def _():
        pltpu.sync_copy(zero_v, acc_sh.at[pl.ds((done + sid) * 16, 16)])

    plsc.subcore_barrier()

    base = sid * _EPC
    nchunk = _EPC // _CH

    def _issue_loads(msg_hbm, c, b):
        off = base + c * _CH
        pltpu.async_copy(dst_hbm.at[pl.ds(off, _CH)], idx_v.at[b], lsem.at[b])
        pltpu.async_copy(msg_hbm.at[pl.ds(off, _CH)], rows_v.at[b], lsem.at[b])

    def _drain_loads(msg_hbm, b):
        pltpu.make_async_copy(dst_hbm.at[pl.ds(0, _CH)], idx_v.at[b], lsem.at[b]).wait()
        pltpu.make_async_copy(msg_hbm.at[pl.ds(0, _CH)], rows_v.at[b], lsem.at[b]).wait()

    def _drain_add(msg_hbm, b):
        pltpu.make_async_copy(msg_hbm.at[pl.ds(0, _CH)], rows_v.at[b], asem.at[b]).wait()

    def _run(msg_hbm):
        for b in range(2):
            _issue_loads(msg_hbm, b, b)

        def _step(i, _):
            for b in range(4):
                c = i * 4 + b
                _drain_loads(msg_hbm, b)
                pltpu.async_copy(rows_v.at[b], acc_sh.at[idx_v.at[b]],
                                 asem.at[b], add=True)
                nc = c + 2
                nb = (b + 2) % 4

                @pl.when(nc >= 4)
                def _():
                    _drain_add(msg_hbm, nb)

                @pl.when(nc < nchunk)
                def _():
                    _issue_loads(msg_hbm, nc, nb)

            return _

        lax.fori_loop(0, nchunk // 4, _step, None)
        _drain_add(msg_hbm, 2)
        _drain_add(msg_hbm, 3)

    @pl.when(cid == 0)
    def _():
        _run(msga_hbm)

    @pl.when(cid == 1)
    def _():
        _run(msgb_hbm)

    plsc.subcore_barrier()

    rows_per = 632

    @pl.when(sid < _SC_NS - 1)
    def _():
        r0 = sid * rows_per
        pltpu.sync_copy(acc_sh.at[pl.ds(r0, rows_per)],
                        out_hbm.at[cid].at[pl.ds(r0, rows_per)])

    @pl.when(sid == _SC_NS - 1)
    def _():
        r0 = (_SC_NS - 1) * rows_per
        pltpu.sync_copy(acc_sh.at[pl.ds(r0, N - r0)],
                        out_hbm.at[cid].at[pl.ds(r0, N - r0)])


def _sc_scatter(msgs_a, msgs_b, dst):
    mesh = plsc.VectorSubcoreMesh(core_axis_name="c", subcore_axis_name="s")
    f = functools.partial(
        pl.kernel,
        mesh=mesh,
        out_type=jax.ShapeDtypeStruct((_SC_NC, N, S), jnp.float32),
        scratch_types=[
            pltpu.VMEM((4, _CH), jnp.int32),
            pltpu.VMEM((4, _CH, S), jnp.float32),
            pltpu.VMEM((16, S), jnp.float32),
            pltpu.VMEM_SHARED((N, S), jnp.float32),
            pltpu.SemaphoreType.DMA((4,)),
            pltpu.SemaphoreType.DMA((4,)),
        ],
    )(_sc_scatter_body)
    return f(msgs_a, msgs_b, dst)


def kernel(scalar_feats, coord_feats, positions, edge_index, params):
    src = edge_index[0]
    dst = edge_index[1]
    cf_packed = jnp.transpose(coord_feats, (0, 2, 1)).reshape(N, 3 * V)

    table = _build_table(scalar_feats, cf_packed,
                         params['msg']['W_out'][:S], params['msg']['Wh'])

    posflat = jnp.pad(positions, ((0, 0), (0, 1))).reshape(-1)
    g, pd = _sc_gather(table, posflat, src, dst)
    msgs_a, msgs_b = _edge_stage(g, pd, params)
    agg = _sc_scatter(msgs_a, msgs_b, dst)

    s2, v2p = _node_stage(scalar_feats, cf_packed, agg, params)
    v2 = jnp.transpose(v2p.reshape(N, 3, V), (0, 2, 1))
    return s2, v2
